# baseline (speedup 1.0000x reference)
import functools

import jax
import jax.numpy as jnp
from jax import lax
from jax.experimental import pallas as pl
from jax.experimental.pallas import tpu as pltpu
from jax.experimental.pallas import tpu_sc as plsc

B = 16384
D = 64
NC = 2
NS = 16
NW = NC * NS
BPW = B // NW
L = 16
NBLK = BPW // L


def _mf_body(uid_hbm, iid_hbm, ut_hbm, it_hbm, ub_hbm, ib_hbm, gb_hbm,
             out_hbm,
             uid_v, iid_v, urows_v, irows_v, ub_v, ib_v, out_v, tbuf_v, gb_v,
             sem):
    wid = lax.axis_index("s") * NC + lax.axis_index("c")
    base = wid * BPW

    pltpu.sync_copy(uid_hbm.at[pl.ds(base, BPW)], uid_v)
    pltpu.sync_copy(iid_hbm.at[pl.ds(base, BPW)], iid_v)
    pltpu.sync_copy(gb_hbm, gb_v)

    cp_u = pltpu.async_copy(ut_hbm.at[uid_v], urows_v, sem)
    cp_i = pltpu.async_copy(it_hbm.at[iid_v], irows_v, sem)
    cp_ub = pltpu.async_copy(ub_hbm.at[uid_v], ub_v, sem)
    cp_ib = pltpu.async_copy(ib_hbm.at[iid_v], ib_v, sem)
    cp_u.wait()
    cp_i.wait()
    cp_ub.wait()
    cp_ib.wait()

    gb = gb_v[...]
    lane = lax.iota(jnp.int32, L)
    col0 = lane * L

    def blk_body(blk, carry):
        del carry
        row0 = blk * L
        for j in range(L):
            r = row0 + j
            acc = urows_v[r, pl.ds(0, L)] * irows_v[r, pl.ds(0, L)]
            for k in range(1, D // L):
                acc = acc + (urows_v[r, pl.ds(k * L, L)]
                             * irows_v[r, pl.ds(k * L, L)])
            tbuf_v[pl.ds(j * L, L)] = acc
        out_vec = ub_v[pl.ds(row0, L)] + ib_v[pl.ds(row0, L)] + gb
        for l in range(L):
            out_vec = out_vec + plsc.load_gather(tbuf_v, [col0 + l])
        out_v[pl.ds(row0, L)] = out_vec
        return 0

    lax.fori_loop(0, NBLK, blk_body, 0)
    pltpu.sync_copy(out_v, out_hbm.at[pl.ds(base, BPW)])


@jax.jit
def _mf(user_ids, item_ids, user_table, item_table, user_bias, item_bias,
        global_bias):
    mesh = plsc.VectorSubcoreMesh(core_axis_name="c", subcore_axis_name="s",
                                  num_cores=NC, num_subcores=NS)
    run = pl.kernel(
        _mf_body,
        out_type=jax.ShapeDtypeStruct((B,), jnp.float32),
        mesh=mesh,
        compiler_params=pltpu.CompilerParams(needs_layout_passes=False,
                                             use_tc_tiling_on_sc=False),
        scratch_types=[
            pltpu.VMEM((BPW,), jnp.int32),
            pltpu.VMEM((BPW,), jnp.int32),
            pltpu.VMEM((BPW, D), jnp.float32),
            pltpu.VMEM((BPW, D), jnp.float32),
            pltpu.VMEM((BPW,), jnp.float32),
            pltpu.VMEM((BPW,), jnp.float32),
            pltpu.VMEM((BPW,), jnp.float32),
            pltpu.VMEM((L * L,), jnp.float32),
            pltpu.VMEM((L,), jnp.float32),
            pltpu.SemaphoreType.DMA,
        ],
    )
    return run(user_ids, item_ids, user_table, item_table, user_bias,
               item_bias, global_bias)


def kernel(user_ids, item_ids, user_table, item_table, user_bias, item_bias,
           global_bias):
    uid = user_ids.astype(jnp.int32)
    iid = item_ids.astype(jnp.int32)
    ub = user_bias.reshape((-1,))
    ib = item_bias.reshape((-1,))
    gb = jnp.broadcast_to(global_bias.reshape(()), (L,))
    return _mf(uid, iid, user_table, item_table, ub, ib, gb)


# speedup vs baseline: 1.1628x; 1.1628x over previous
import functools

import jax
import jax.numpy as jnp
from jax import lax
from jax.experimental import pallas as pl
from jax.experimental.pallas import tpu as pltpu
from jax.experimental.pallas import tpu_sc as plsc

B = 16384
D = 64
NC = 2
NS = 16
NW = NC * NS
BPW = B // NW
L = 16
CH = 64
NCH = BPW // CH


def _mf_body(uid_hbm, iid_hbm, ut_hbm, it_hbm, ub_hbm, ib_hbm, gb_hbm,
             out_hbm,
             uid_v, iid_v, urows_v, irows_v, ub_v, ib_v, out_v, tbuf_v, gb_v,
             sem, sem_rows):
    wid = lax.axis_index("s") * NC + lax.axis_index("c")
    base = wid * BPW

    pltpu.sync_copy(uid_hbm.at[pl.ds(base, BPW)], uid_v)
    pltpu.sync_copy(iid_hbm.at[pl.ds(base, BPW)], iid_v)
    pltpu.sync_copy(gb_hbm, gb_v)

    cp_ub = pltpu.async_copy(ub_hbm.at[uid_v], ub_v, sem)
    cp_ib = pltpu.async_copy(ib_hbm.at[iid_v], ib_v, sem)
    cp_ub.wait()
    cp_ib.wait()

    gb = gb_v[...]
    lane = lax.iota(jnp.int32, L)
    col0 = lane * L

    def chunk_body(g, carry):
        del carry
        row0 = g * CH
        cps = []
        for jj in range(CH // L):
            uvec = uid_v[pl.ds(row0 + jj * L, L)]
            ivec = iid_v[pl.ds(row0 + jj * L, L)]
            for j in range(L):
                r = row0 + jj * L + j
                cps.append(pltpu.async_copy(
                    ut_hbm.at[uvec[j]], urows_v.at[jj * L + j], sem_rows))
                cps.append(pltpu.async_copy(
                    it_hbm.at[ivec[j]], irows_v.at[jj * L + j], sem_rows))
        for cp in cps:
            cp.wait()
        for jj in range(CH // L):
            for j in range(L):
                rr = jj * L + j
                acc = (urows_v[rr, pl.ds(0, L)] * irows_v[rr, pl.ds(0, L)])
                for k in range(1, D // L):
                    acc = acc + (urows_v[rr, pl.ds(k * L, L)]
                                 * irows_v[rr, pl.ds(k * L, L)])
                tbuf_v[pl.ds(j * L, L)] = acc
            out_vec = (ub_v[pl.ds(row0 + jj * L, L)]
                       + ib_v[pl.ds(row0 + jj * L, L)] + gb)
            for l in range(L):
                out_vec = out_vec + plsc.load_gather(tbuf_v, [col0 + l])
            out_v[pl.ds(row0 + jj * L, L)] = out_vec
        return 0

    lax.fori_loop(0, NCH, chunk_body, 0)
    pltpu.sync_copy(out_v, out_hbm.at[pl.ds(base, BPW)])


@jax.jit
def _mf(user_ids, item_ids, user_table, item_table, user_bias, item_bias,
        global_bias):
    mesh = plsc.VectorSubcoreMesh(core_axis_name="c", subcore_axis_name="s",
                                  num_cores=NC, num_subcores=NS)
    run = pl.kernel(
        _mf_body,
        out_type=jax.ShapeDtypeStruct((B,), jnp.float32),
        mesh=mesh,
        compiler_params=pltpu.CompilerParams(needs_layout_passes=False,
                                             use_tc_tiling_on_sc=True),
        scratch_types=[
            pltpu.VMEM((BPW,), jnp.int32),
            pltpu.VMEM((BPW,), jnp.int32),
            pltpu.VMEM((CH, D), jnp.float32),
            pltpu.VMEM((CH, D), jnp.float32),
            pltpu.VMEM((BPW,), jnp.float32),
            pltpu.VMEM((BPW,), jnp.float32),
            pltpu.VMEM((BPW,), jnp.float32),
            pltpu.VMEM((L * L,), jnp.float32),
            pltpu.VMEM((L,), jnp.float32),
            pltpu.SemaphoreType.DMA,
            pltpu.SemaphoreType.DMA,
        ],
    )
    return run(user_ids, item_ids, user_table, item_table, user_bias,
               item_bias, global_bias)


def kernel(user_ids, item_ids, user_table, item_table, user_bias, item_bias,
           global_bias):
    uid = user_ids.astype(jnp.int32)
    iid = item_ids.astype(jnp.int32)
    ub = user_bias.reshape((-1,))
    ib = item_bias.reshape((-1,))
    gb = jnp.broadcast_to(global_bias.reshape(()), (L,))
    return _mf(uid, iid, user_table, item_table, ub, ib, gb)
